# baseline (speedup 1.0000x reference)
import functools

import jax
import jax.numpy as jnp
from jax import lax
from jax.experimental import pallas as pl
from jax.experimental.pallas import tpu as pltpu
from jax.experimental.pallas import tpu_sc as plsc

B = 8192
N = 64
F = 16
H = 64
C = 128

BL = 512
SC_SHARE = 4096
TC_SHARE = B - SC_SHARE
OFF_BLK = TC_SHARE // BL
LG = 128


def _f32dot(a, b, dims):
    return lax.dot_general(a, b, (dims, ((), ())),
                           preferred_element_type=jnp.float32)


def _sc_ego_rows(adjT, aid_sc):
    info = plsc.get_sparse_core_info()
    mesh = plsc.VectorSubcoreMesh(core_axis_name="c", subcore_axis_name="s")

    @functools.partial(
        pl.kernel,
        mesh=mesh,
        out_type=jax.ShapeDtypeStruct((N, SC_SHARE), jnp.float32),
        scratch_types=[
            pltpu.VMEM((LG,), jnp.int32),
            pltpu.VMEM((2, N, LG), jnp.float32),
            pltpu.VMEM((N, LG), jnp.float32),
            pltpu.SemaphoreType.DMA,
            pltpu.SemaphoreType.DMA,
        ],
    )
    def sc_kernel(adjT_hbm, aid_hbm, out_hbm, aid_v, slab_v, acc_v, sem0, sem1):
        w = lax.axis_index("s") * info.num_cores + lax.axis_index("c")
        gl = w * LG
        pltpu.sync_copy(aid_hbm.at[pl.ds(gl, LG)], aid_v)
        zero = jnp.zeros((16,), jnp.float32)
        for r in range(N):
            for l in range(LG // 16):
                acc_v[r, pl.ds(l * 16, 16)] = zero

        def fire(n1, slot, sem):
            pltpu.async_copy(
                adjT_hbm.at[n1, :, pl.ds(TC_SHARE + gl, LG)], slab_v.at[slot], sem
            )

        def take(n1, slot):
            for l in range(LG // 16):
                m16 = aid_v[pl.ds(l * 16, 16)] == n1
                for r in range(N):
                    sl = slab_v[slot, r, pl.ds(l * 16, 16)]
                    acc_v[r, pl.ds(l * 16, 16)] = jnp.where(
                        m16, sl, acc_v[r, pl.ds(l * 16, 16)]
                    )

        fire(0, 0, sem0)

        def step(i, carry):
            n1 = 2 * i
            fire(jnp.minimum(n1 + 1, N - 1), 1, sem1)
            pltpu.make_async_copy(
                adjT_hbm.at[0, :, pl.ds(0, LG)], slab_v.at[0], sem0
            ).wait()
            take(n1, 0)
            fire(jnp.minimum(n1 + 2, N - 1), 0, sem0)
            pltpu.make_async_copy(
                adjT_hbm.at[0, :, pl.ds(0, LG)], slab_v.at[1], sem1
            ).wait()
            take(n1 + 1, 1)
            return carry

        lax.fori_loop(0, N // 2, step, 0)
        pltpu.make_async_copy(
            adjT_hbm.at[0, :, pl.ds(0, LG)], slab_v.at[0], sem0
        ).wait()
        pltpu.sync_copy(acc_v, out_hbm.at[:, pl.ds(gl, LG)])

    return sc_kernel(adjT, aid_sc)


def _gnn_mlp(aT, mask, nobsT_ref, cent_ref,
             we_ref, be_ref, wms_ref,
             w1c_ref, w1h_ref, b1_ref, w2_ref, b2_ref, wv_ref, bv_ref,
             out_ref):
    f32 = jnp.float32
    nobsT = nobsT_ref[...]
    we = we_ref[...]
    be = be_ref[...]
    m = jnp.zeros((H, BL), f32)
    nobs_ego = jnp.zeros((F, BL), f32)
    for n in range(N):
        h_n = jax.nn.relu(_f32dot(we, nobsT[n], ((0,), (0,))) + be)
        m = m + aT[n:n + 1, :] * h_n
        nobs_ego = nobs_ego + mask[n:n + 1, :] * nobsT[n]
    ego = jax.nn.relu(_f32dot(we, nobs_ego, ((0,), (0,))) + be)

    p = jnp.concatenate([m, ego], axis=0)
    nbd = jax.nn.relu(_f32dot(p, wms_ref[...], ((0,), (0,))))

    x = jax.nn.relu(
        _f32dot(cent_ref[...], w1c_ref[...], ((1,), (0,)))
        + _f32dot(nbd, w1h_ref[...], ((1,), (0,)))
        + b1_ref[...]
    )
    x = jax.nn.relu(_f32dot(x, w2_ref[...], ((1,), (0,))) + b2_ref[...])
    out_ref[...] = jnp.sum(x * wv_ref[...], axis=1, keepdims=True) + bv_ref[...]


def _mask_of(aid_ref):
    aid = aid_ref[...]
    return (lax.broadcasted_iota(jnp.int32, (N, BL), 0) == aid).astype(jnp.float32)


def _tc_body_a(adjT_ref, nobsT_ref, aidT_ref, cent_ref, *rest):
    mask = _mask_of(aidT_ref)
    adjT = adjT_ref[...]
    arow = jnp.zeros((N, BL), jnp.float32)
    for n1 in range(N):
        arow = arow + mask[n1:n1 + 1, :] * adjT[n1]
    deg = jnp.sum(arow, axis=0, keepdims=True)
    aT = arow / (deg + 1e-6)
    _gnn_mlp(aT, mask, nobsT_ref, cent_ref, *rest)


def _tc_body_b(arow_ref, nobsT_ref, aidT_ref, cent_ref, *rest):
    mask = _mask_of(aidT_ref)
    arow = arow_ref[...]
    deg = jnp.sum(arow, axis=0, keepdims=True)
    aT = arow / (deg + 1e-6)
    _gnn_mlp(aT, mask, nobsT_ref, cent_ref, *rest)


def kernel(cent_obs, node_obs, adj, agent_id, rnn_states, masks,
           W_embed, b_embed, W_msg, W_self, W1, b1, W2, b2, Wv, bv):
    adjT = jnp.transpose(adj, (1, 2, 0))
    nobsT = jnp.transpose(node_obs, (1, 2, 0))
    aid32 = agent_id.astype(jnp.int32)
    aidT = aid32.reshape(1, B)
    wms = jnp.concatenate([W_msg, W_self], axis=0)

    arow_sc = _sc_ego_rows(adjT, aid32[TC_SHARE:, 0])

    weights = (W_embed, b_embed.reshape(H, 1), wms,
               W1[:C], W1[C:], b1.reshape(1, H), W2, b2.reshape(1, H),
               Wv.reshape(1, H), bv.reshape(1, 1))
    wspecs = [
        pl.BlockSpec(s, lambda i: (0,) * len(s))
        for s in [(F, H), (H, 1), (2 * H, H), (C, H), (H, H), (1, H),
                  (H, H), (1, H), (1, H), (1, 1)]
    ]

    vals_a = pl.pallas_call(
        _tc_body_a,
        grid=(TC_SHARE // BL,),
        in_specs=[
            pl.BlockSpec((N, N, BL), lambda i: (0, 0, i)),
            pl.BlockSpec((N, F, BL), lambda i: (0, 0, i)),
            pl.BlockSpec((1, BL), lambda i: (0, i)),
            pl.BlockSpec((BL, C), lambda i: (i, 0)),
            *wspecs,
        ],
        out_specs=pl.BlockSpec((BL, 1), lambda i: (i, 0)),
        out_shape=jax.ShapeDtypeStruct((TC_SHARE, 1), jnp.float32),
    )(adjT, nobsT, aidT, cent_obs, *weights)

    vals_b = pl.pallas_call(
        _tc_body_b,
        grid=(SC_SHARE // BL,),
        in_specs=[
            pl.BlockSpec((N, BL), lambda i: (0, i)),
            pl.BlockSpec((N, F, BL), lambda i: (0, 0, i + OFF_BLK)),
            pl.BlockSpec((1, BL), lambda i: (0, i + OFF_BLK)),
            pl.BlockSpec((BL, C), lambda i: (i + OFF_BLK, 0)),
            *wspecs,
        ],
        out_specs=pl.BlockSpec((BL, 1), lambda i: (i, 0)),
        out_shape=jax.ShapeDtypeStruct((SC_SHARE, 1), jnp.float32),
    )(arow_sc, nobsT, aidT, cent_obs, *weights)

    values = jnp.concatenate([vals_a, vals_b], axis=0)
    return values, rnn_states


# speedup vs baseline: 2.6698x; 2.6698x over previous
import jax
import jax.numpy as jnp
from jax import lax
from jax.experimental import pallas as pl

B = 8192
N = 64
F = 16
H = 64
C = 128

BL = 512


def _f32dot(a, b, dims):
    return lax.dot_general(a, b, (dims, ((), ())),
                           preferred_element_type=jnp.float32)


def _tc_body(adjT_ref, nobsT_ref, aidT_ref, cent_ref,
             we_ref, be_ref, wms_ref,
             w1c_ref, w1h_ref, b1_ref, w2_ref, b2_ref, wv_ref, bv_ref,
             out_ref):
    f32 = jnp.float32
    aid = aidT_ref[...]
    mask = (lax.broadcasted_iota(jnp.int32, (N, BL), 0) == aid).astype(f32)

    adjT = adjT_ref[...]
    arow = jnp.zeros((N, BL), f32)
    for n1 in range(N):
        arow = arow + mask[n1:n1 + 1, :] * adjT[n1]
    deg = jnp.sum(arow, axis=0, keepdims=True)
    aT = arow / (deg + 1e-6)

    nobsT = nobsT_ref[...]
    we = we_ref[...]
    be = be_ref[...]
    m = jnp.zeros((H, BL), f32)
    nobs_ego = jnp.zeros((F, BL), f32)
    for n in range(N):
        h_n = jax.nn.relu(_f32dot(we, nobsT[n], ((0,), (0,))) + be)
        m = m + aT[n:n + 1, :] * h_n
        nobs_ego = nobs_ego + mask[n:n + 1, :] * nobsT[n]
    ego = jax.nn.relu(_f32dot(we, nobs_ego, ((0,), (0,))) + be)

    p = jnp.concatenate([m, ego], axis=0)
    nbd = jax.nn.relu(_f32dot(p, wms_ref[...], ((0,), (0,))))

    x = jax.nn.relu(
        _f32dot(cent_ref[...], w1c_ref[...], ((1,), (0,)))
        + _f32dot(nbd, w1h_ref[...], ((1,), (0,)))
        + b1_ref[...]
    )
    x = jax.nn.relu(_f32dot(x, w2_ref[...], ((1,), (0,))) + b2_ref[...])
    out_ref[...] = jnp.sum(x * wv_ref[...], axis=1, keepdims=True) + bv_ref[...]


def kernel(cent_obs, node_obs, adj, agent_id, rnn_states, masks,
           W_embed, b_embed, W_msg, W_self, W1, b1, W2, b2, Wv, bv):
    adjT = jnp.transpose(adj, (1, 2, 0))
    nobsT = jnp.transpose(node_obs, (1, 2, 0))
    aidT = agent_id.astype(jnp.int32).reshape(1, B)
    wms = jnp.concatenate([W_msg, W_self], axis=0)

    grid = (B // BL,)
    full = lambda *s: pl.BlockSpec(s, lambda i: (0,) * len(s))
    values = pl.pallas_call(
        _tc_body,
        grid=grid,
        in_specs=[
            pl.BlockSpec((N, N, BL), lambda i: (0, 0, i)),
            pl.BlockSpec((N, F, BL), lambda i: (0, 0, i)),
            pl.BlockSpec((1, BL), lambda i: (0, i)),
            pl.BlockSpec((BL, C), lambda i: (i, 0)),
            full(F, H),
            full(H, 1),
            full(2 * H, H),
            full(C, H),
            full(H, H),
            full(1, H),
            full(H, H),
            full(1, H),
            full(1, H),
            full(1, 1),
        ],
        out_specs=pl.BlockSpec((BL, 1), lambda i: (i, 0)),
        out_shape=jax.ShapeDtypeStruct((B, 1), jnp.float32),
    )(adjT, nobsT, aidT, cent_obs,
      W_embed, b_embed.reshape(H, 1), wms,
      W1[:C], W1[C:], b1.reshape(1, H), W2, b2.reshape(1, H),
      Wv.reshape(1, H), bv.reshape(1, 1))
    return values, rnn_states
